# Initial kernel scaffold; baseline (speedup 1.0000x reference)
import jax
import jax.numpy as jnp
from jax.experimental import pallas as pl


def kernel(nfeat, perm_idx, perm_vals, pool_seg_ids, pool_vals, degs, weights, bias, W0, b0, W1, b1):
    raise NotImplementedError("write your pallas kernel here")


# speedup vs baseline: 21.4648x; 21.4648x over previous
import functools

import jax
import jax.numpy as jnp
from jax import lax
from jax.experimental import pallas as pl
from jax.experimental.pallas import tpu as pltpu
from jax.experimental.pallas import tpu_sc as plsc

_NCORES = 2
_NSUB = 16
_WIN = 8192
_CHUNK = 128


def _sc_gather(nfeat_t, perm_idx):
    n = nfeat_t.shape[1]
    m = perm_idx.shape[0]
    per_sub = m // _NSUB
    ch = 4000
    n_ch = per_sub // ch
    assert per_sub % ch == 0

    mesh = plsc.VectorSubcoreMesh(core_axis_name="c", subcore_axis_name="s")

    @functools.partial(
        pl.kernel,
        mesh=mesh,
        out_type=[jax.ShapeDtypeStruct((m,), jnp.float32),
                  jax.ShapeDtypeStruct((m,), jnp.float32)],
        scratch_types=[pltpu.VMEM((n,), jnp.float32),
                       pltpu.VMEM((ch,), jnp.int32),
                       pltpu.VMEM((ch,), jnp.float32)],
        compiler_params=pltpu.CompilerParams(needs_layout_passes=False),
    )
    def k(nfeat_hbm, idx_hbm, g0_hbm, g1_hbm, table_v, idx_v, out_v):
        c = lax.axis_index("c")
        s = lax.axis_index("s")
        pltpu.sync_copy(nfeat_hbm.at[c], table_v)
        base = s * per_sub

        def chunk_body(j, carry):
            off = base + j * ch
            pltpu.sync_copy(idx_hbm.at[pl.ds(off, ch)], idx_v)

            def gbody(i, carry2):
                idxv = idx_v[pl.ds(i * 16, 16)]
                out_v[pl.ds(i * 16, 16)] = plsc.load_gather(table_v, [idxv])
                return carry2

            lax.fori_loop(0, ch // 16, gbody, 0, unroll=4)

            @pl.when(c == 0)
            def _():
                pltpu.sync_copy(out_v, g0_hbm.at[pl.ds(off, ch)])

            @pl.when(c == 1)
            def _():
                pltpu.sync_copy(out_v, g1_hbm.at[pl.ds(off, ch)])

            return carry

        lax.fori_loop(0, n_ch, chunk_body, 0)

    return k(nfeat_t, perm_idx)


def _tc_matmul(g0, g1, vals, pool_vals, w_cat, bias):
    p, l = g0.shape
    dout = w_cat.shape[1]
    bp = 2000
    assert p % bp == 0

    def body(g0_r, g1_r, v_r, pv_r, w_r, b_r, o_r):
        v = v_r[...]
        x = jnp.concatenate([g0_r[...] * v, g1_r[...] * v], axis=-1)
        h = jnp.dot(x, w_r[...], preferred_element_type=jnp.float32)
        h = jnp.maximum(h + b_r[...], 0.0)
        o_r[...] = h * pv_r[...]

    return pl.pallas_call(
        body,
        grid=(p // bp,),
        in_specs=[pl.BlockSpec((bp, l), lambda i: (i, 0)),
                  pl.BlockSpec((bp, l), lambda i: (i, 0)),
                  pl.BlockSpec((bp, l), lambda i: (i, 0)),
                  pl.BlockSpec((bp, 1), lambda i: (i, 0)),
                  pl.BlockSpec((2 * l, dout), lambda i: (0, 0)),
                  pl.BlockSpec((1, dout), lambda i: (0, 0))],
        out_specs=pl.BlockSpec((bp, dout), lambda i: (i, 0)),
        out_shape=jax.ShapeDtypeStruct((p, dout), jnp.float32),
    )(g0, g1, vals, pool_vals, w_cat, bias)


def _sc_segsum(hs, seg, bounds, n_nodes):
    p, dout = hs.shape
    n_win = -(-n_nodes // _WIN)
    zrows = 520
    wbuf_rows = zrows * _NSUB
    assert p % _CHUNK == 0

    mesh = plsc.VectorSubcoreMesh(core_axis_name="c", subcore_axis_name="s")

    @functools.partial(
        pl.kernel,
        mesh=mesh,
        out_type=jax.ShapeDtypeStruct((n_nodes, dout), jnp.float32),
        scratch_types=[pltpu.VMEM((_CHUNK, dout), jnp.float32),
                       pltpu.VMEM((_CHUNK,), jnp.int32),
                       pltpu.VMEM((_CHUNK,), jnp.int32),
                       pltpu.VMEM((16,), jnp.int32),
                       pltpu.VMEM((_CHUNK, dout), jnp.float32),
                       pltpu.VMEM_SHARED((wbuf_rows, dout), jnp.float32)],
        compiler_params=pltpu.CompilerParams(needs_layout_passes=False),
    )
    def k(hs_hbm, seg_hbm, bounds_hbm, out_hbm,
          rows_v, seg_v, idx_v, bounds_v, zbuf_v, win_sh):
        c = lax.axis_index("c")
        s = lax.axis_index("s")
        pltpu.sync_copy(bounds_hbm, bounds_v)
        bvec = bounds_v[...]
        iot = lax.iota(jnp.int32, 16)

        def zb(i, carry):
            for u in range(dout // 16):
                zbuf_v[i, pl.ds(u * 16, 16)] = jnp.zeros((16,), jnp.float32)
            return carry

        lax.fori_loop(0, _CHUNK, zb, 0)

        def extract(idx_static):
            return jnp.max(jnp.where(iot == idx_static, bvec,
                                     jnp.int32(-2147483648)))

        for w in range(n_win):
            n0 = w * _WIN
            wn = min(_WIN, n_nodes - n0)

            @pl.when(c == (w % _NCORES))
            def _(w=w, n0=n0, wn=wn):
                r0 = extract(w)
                r1 = extract(w + 1)
                nfull, ztail = divmod(zrows, _CHUNK)
                for q in range(nfull):
                    pltpu.sync_copy(
                        zbuf_v,
                        win_sh.at[pl.ds(s * zrows + q * _CHUNK, _CHUNK)])
                if ztail:
                    pltpu.sync_copy(
                        zbuf_v.at[pl.ds(0, ztail)],
                        win_sh.at[pl.ds(s * zrows + nfull * _CHUNK, ztail)])
                plsc.subcore_barrier()

                k0 = r0 // _CHUNK
                k1 = (r1 + _CHUNK - 1) // _CHUNK
                nk = jnp.maximum(0, (k1 - k0 - s + _NSUB - 1) // _NSUB)

                def chunk(t, carry):
                    kk = k0 + s + t * _NSUB
                    rbase = kk * _CHUNK
                    pltpu.sync_copy(hs_hbm.at[pl.ds(rbase, _CHUNK)], rows_v)
                    pltpu.sync_copy(seg_hbm.at[pl.ds(rbase, _CHUNK)], seg_v)
                    for u in range(_CHUNK // 16):
                        sv = seg_v[pl.ds(u * 16, 16)]
                        absrow = rbase + u * 16 + iot
                        valid = (absrow >= r0) & (absrow < r1)
                        li = jnp.where(valid, sv - n0,
                                       jnp.int32(_WIN) + s)
                        idx_v[pl.ds(u * 16, 16)] = li
                    pltpu.sync_copy(rows_v, win_sh.at[idx_v], add=True)
                    return carry

                lax.fori_loop(0, nk, chunk, 0)
                plsc.subcore_barrier()

                share = (wn // _NSUB) & ~7
                if share > 0:
                    pltpu.sync_copy(
                        win_sh.at[pl.ds(s * share, share)],
                        out_hbm.at[pl.ds(n0 + s * share, share)])
                rem = wn - share * _NSUB
                if rem > 0:
                    @pl.when(s == 0)
                    def _(share=share, rem=rem, n0=n0):
                        pltpu.sync_copy(
                            win_sh.at[pl.ds(share * _NSUB, rem)],
                            out_hbm.at[pl.ds(n0 + share * _NSUB, rem)])
                plsc.subcore_barrier()

    return k(hs, seg, bounds)


def _tc_finish(pooled, degs2d, w0t, b0, w1t, b1):
    n, dout = pooled.shape
    dh = w0t.shape[1]
    bn = 2000
    assert n % bn == 0

    def body(p_r, d_r, w0_r, b0_r, w1_r, b1_r, o_r):
        f1 = jnp.maximum(d_r[...] * w0_r[...] + b0_r[...], 0.0)
        f = jnp.dot(f1, w1_r[...], preferred_element_type=jnp.float32)
        f = f + b1_r[...]
        o_r[...] = jnp.maximum(p_r[...] * f, 0.0)

    return pl.pallas_call(
        body,
        grid=(n // bn,),
        in_specs=[pl.BlockSpec((bn, dout), lambda i: (i, 0)),
                  pl.BlockSpec((bn, 1), lambda i: (i, 0)),
                  pl.BlockSpec((1, dh), lambda i: (0, 0)),
                  pl.BlockSpec((1, dh), lambda i: (0, 0)),
                  pl.BlockSpec((dh, dout), lambda i: (0, 0)),
                  pl.BlockSpec((1, dout), lambda i: (0, 0))],
        out_specs=pl.BlockSpec((bn, dout), lambda i: (i, 0)),
        out_shape=jax.ShapeDtypeStruct((n, dout), jnp.float32),
    )(pooled, degs2d, w0t, b0, w1t, b1)


def kernel(nfeat, perm_idx, perm_vals, pool_seg_ids, pool_vals, degs,
           weights, bias, W0, b0, W1, b1):
    n_nodes, d_in = nfeat.shape
    l = weights.shape[2]
    dout = weights.shape[1]
    m = perm_idx.shape[0]
    p = m // l

    g0, g1 = _sc_gather(jnp.transpose(nfeat), perm_idx)

    w_cat = jnp.concatenate([jnp.transpose(weights[0]),
                             jnp.transpose(weights[1])], axis=0)
    hs = _tc_matmul(g0.reshape(p, l), g1.reshape(p, l),
                    perm_vals.reshape(p, l), pool_vals.reshape(p, 1),
                    w_cat, bias)

    n_win = -(-n_nodes // _WIN)
    cuts = jnp.arange(1, n_win + 1, dtype=jnp.int32) * _WIN
    bounds = jnp.searchsorted(pool_seg_ids, cuts, side="left").astype(jnp.int32)
    bounds = jnp.concatenate([jnp.zeros((1,), jnp.int32), bounds])
    bounds = jnp.pad(bounds, (0, 16 - bounds.shape[0]),
                     constant_values=jnp.int32(m // l))
    pooled = _sc_segsum(hs, pool_seg_ids, bounds, n_nodes)

    out = _tc_finish(pooled, degs[:, None], jnp.transpose(W0), b0[None, :],
                     jnp.transpose(W1), b1[None, :])
    return out


# speedup vs baseline: 30.1973x; 1.4068x over previous
import functools

import jax
import jax.numpy as jnp
from jax import lax
from jax.experimental import pallas as pl
from jax.experimental.pallas import tpu as pltpu
from jax.experimental.pallas import tpu_sc as plsc

_NCORES = 2
_NSUB = 16
_WIN = 8192
_CHUNK = 128


def _sc_gather(nfeat_t, perm_idx):
    n = nfeat_t.shape[1]
    m = perm_idx.shape[0]
    per_sub = m // _NSUB
    ch = 4000
    n_ch = per_sub // ch
    assert per_sub % ch == 0

    mesh = plsc.VectorSubcoreMesh(core_axis_name="c", subcore_axis_name="s")

    @functools.partial(
        pl.kernel,
        mesh=mesh,
        out_type=[jax.ShapeDtypeStruct((m,), jnp.float32),
                  jax.ShapeDtypeStruct((m,), jnp.float32)],
        scratch_types=[pltpu.VMEM((n,), jnp.float32),
                       pltpu.VMEM((ch,), jnp.int32),
                       pltpu.VMEM((ch,), jnp.float32)],
        compiler_params=pltpu.CompilerParams(needs_layout_passes=False),
    )
    def k(nfeat_hbm, idx_hbm, g0_hbm, g1_hbm, table_v, idx_v, out_v):
        c = lax.axis_index("c")
        s = lax.axis_index("s")
        pltpu.sync_copy(nfeat_hbm.at[c], table_v)
        base = s * per_sub

        def chunk_body(j, carry):
            off = base + j * ch
            pltpu.sync_copy(idx_hbm.at[pl.ds(off, ch)], idx_v)

            def gbody(i, carry2):
                idxv = idx_v[pl.ds(i * 16, 16)]
                out_v[pl.ds(i * 16, 16)] = plsc.load_gather(table_v, [idxv])
                return carry2

            lax.fori_loop(0, ch // 16, gbody, 0, unroll=4)

            @pl.when(c == 0)
            def _():
                pltpu.sync_copy(out_v, g0_hbm.at[pl.ds(off, ch)])

            @pl.when(c == 1)
            def _():
                pltpu.sync_copy(out_v, g1_hbm.at[pl.ds(off, ch)])

            return carry

        lax.fori_loop(0, n_ch, chunk_body, 0)

    return k(nfeat_t, perm_idx)


def _tc_matmul(g0, g1, vals, pool_vals, w0_big, w1_big, bias_wide, p, l, dout):
    bp = 3200
    assert p % bp == 0
    fl = bp * l
    rows = fl // 128

    def body(g0_r, g1_r, v_r, pv_r, w0_r, w1_r, b_r, o_r):
        v = v_r[...]
        x0 = g0_r[...] * v
        x1 = g1_r[...] * v
        hw = jnp.dot(x0, w0_r[...], preferred_element_type=jnp.float32)
        hw += jnp.dot(x1, w1_r[...], preferred_element_type=jnp.float32)
        hw = jnp.maximum(hw + b_r[...], 0.0)
        o_r[...] = hw.reshape(bp, dout) * pv_r[...]

    g0v = g0.reshape(p * l // 128, 128)
    g1v = g1.reshape(p * l // 128, 128)
    vv = vals.reshape(p * l // 128, 128)
    return pl.pallas_call(
        body,
        grid=(p // bp,),
        in_specs=[pl.BlockSpec((rows, 128), lambda i: (i, 0)),
                  pl.BlockSpec((rows, 128), lambda i: (i, 0)),
                  pl.BlockSpec((rows, 128), lambda i: (i, 0)),
                  pl.BlockSpec((bp, 1), lambda i: (i, 0)),
                  pl.BlockSpec((128, 8 * dout), lambda i: (0, 0)),
                  pl.BlockSpec((128, 8 * dout), lambda i: (0, 0)),
                  pl.BlockSpec((1, 8 * dout), lambda i: (0, 0))],
        out_specs=pl.BlockSpec((bp, dout), lambda i: (i, 0)),
        out_shape=jax.ShapeDtypeStruct((p, dout), jnp.float32),
    )(g0v, g1v, vv, pool_vals, w0_big, w1_big, bias_wide)


def _sc_segsum(hs, seg, bounds, n_nodes):
    p, dout = hs.shape
    n_win = -(-n_nodes // _WIN)
    zrows = 520
    wbuf_rows = zrows * _NSUB
    assert p % _CHUNK == 0

    mesh = plsc.VectorSubcoreMesh(core_axis_name="c", subcore_axis_name="s")

    @functools.partial(
        pl.kernel,
        mesh=mesh,
        out_type=jax.ShapeDtypeStruct((n_nodes, dout), jnp.float32),
        scratch_types=[pltpu.VMEM((_CHUNK, dout), jnp.float32),
                       pltpu.VMEM((_CHUNK,), jnp.int32),
                       pltpu.VMEM((_CHUNK,), jnp.int32),
                       pltpu.VMEM((16,), jnp.int32),
                       pltpu.VMEM((_CHUNK, dout), jnp.float32),
                       pltpu.VMEM_SHARED((wbuf_rows, dout), jnp.float32)],
        compiler_params=pltpu.CompilerParams(needs_layout_passes=False),
    )
    def k(hs_hbm, seg_hbm, bounds_hbm, out_hbm,
          rows_v, seg_v, idx_v, bounds_v, zbuf_v, win_sh):
        c = lax.axis_index("c")
        s = lax.axis_index("s")
        pltpu.sync_copy(bounds_hbm, bounds_v)
        bvec = bounds_v[...]
        iot = lax.iota(jnp.int32, 16)

        def zb(i, carry):
            for u in range(dout // 16):
                zbuf_v[i, pl.ds(u * 16, 16)] = jnp.zeros((16,), jnp.float32)
            return carry

        lax.fori_loop(0, _CHUNK, zb, 0)

        def extract(idx_static):
            return jnp.max(jnp.where(iot == idx_static, bvec,
                                     jnp.int32(-2147483648)))

        for w in range(n_win):
            n0 = w * _WIN
            wn = min(_WIN, n_nodes - n0)

            @pl.when(c == (w % _NCORES))
            def _(w=w, n0=n0, wn=wn):
                r0 = extract(w)
                r1 = extract(w + 1)
                nfull, ztail = divmod(zrows, _CHUNK)
                for q in range(nfull):
                    pltpu.sync_copy(
                        zbuf_v,
                        win_sh.at[pl.ds(s * zrows + q * _CHUNK, _CHUNK)])
                if ztail:
                    pltpu.sync_copy(
                        zbuf_v.at[pl.ds(0, ztail)],
                        win_sh.at[pl.ds(s * zrows + nfull * _CHUNK, ztail)])
                plsc.subcore_barrier()

                k0 = r0 // _CHUNK
                k1 = (r1 + _CHUNK - 1) // _CHUNK
                nk = jnp.maximum(0, (k1 - k0 - s + _NSUB - 1) // _NSUB)

                def chunk(t, carry):
                    kk = k0 + s + t * _NSUB
                    rbase = kk * _CHUNK
                    pltpu.sync_copy(hs_hbm.at[pl.ds(rbase, _CHUNK)], rows_v)
                    pltpu.sync_copy(seg_hbm.at[pl.ds(rbase, _CHUNK)], seg_v)
                    for u in range(_CHUNK // 16):
                        sv = seg_v[pl.ds(u * 16, 16)]
                        absrow = rbase + u * 16 + iot
                        valid = (absrow >= r0) & (absrow < r1)
                        li = jnp.where(valid, sv - n0,
                                       jnp.int32(_WIN) + s)
                        idx_v[pl.ds(u * 16, 16)] = li
                    pltpu.sync_copy(rows_v, win_sh.at[idx_v], add=True)
                    return carry

                lax.fori_loop(0, nk, chunk, 0)
                plsc.subcore_barrier()

                share = (wn // _NSUB) & ~7
                if share > 0:
                    pltpu.sync_copy(
                        win_sh.at[pl.ds(s * share, share)],
                        out_hbm.at[pl.ds(n0 + s * share, share)])
                rem = wn - share * _NSUB
                if rem > 0:
                    @pl.when(s == 0)
                    def _(share=share, rem=rem, n0=n0):
                        pltpu.sync_copy(
                            win_sh.at[pl.ds(share * _NSUB, rem)],
                            out_hbm.at[pl.ds(n0 + share * _NSUB, rem)])
                plsc.subcore_barrier()

    return k(hs, seg, bounds)


def _tc_finish(pooled, degs2d, w0t, b0, w1t, b1):
    n, dout = pooled.shape
    dh = w0t.shape[1]
    bn = 2000
    assert n % bn == 0

    def body(p_r, d_r, w0_r, b0_r, w1_r, b1_r, o_r):
        f1 = jnp.maximum(d_r[...] * w0_r[...] + b0_r[...], 0.0)
        f = jnp.dot(f1, w1_r[...], preferred_element_type=jnp.float32)
        f = f + b1_r[...]
        o_r[...] = jnp.maximum(p_r[...] * f, 0.0)

    return pl.pallas_call(
        body,
        grid=(n // bn,),
        in_specs=[pl.BlockSpec((bn, dout), lambda i: (i, 0)),
                  pl.BlockSpec((bn, 1), lambda i: (i, 0)),
                  pl.BlockSpec((1, dh), lambda i: (0, 0)),
                  pl.BlockSpec((1, dh), lambda i: (0, 0)),
                  pl.BlockSpec((dh, dout), lambda i: (0, 0)),
                  pl.BlockSpec((1, dout), lambda i: (0, 0))],
        out_specs=pl.BlockSpec((bn, dout), lambda i: (i, 0)),
        out_shape=jax.ShapeDtypeStruct((n, dout), jnp.float32),
    )(pooled, degs2d, w0t, b0, w1t, b1)


def kernel(nfeat, perm_idx, perm_vals, pool_seg_ids, pool_vals, degs,
           weights, bias, W0, b0, W1, b1):
    n_nodes, d_in = nfeat.shape
    l = weights.shape[2]
    dout = weights.shape[1]
    m = perm_idx.shape[0]
    p = m // l

    g0, g1 = _sc_gather(jnp.transpose(nfeat), perm_idx)

    eye8 = jnp.eye(8, dtype=jnp.float32)
    w0_big = (eye8[:, None, :, None]
              * jnp.transpose(weights[0])[None, :, None, :]).reshape(
                  8 * l, 8 * dout)
    w1_big = (eye8[:, None, :, None]
              * jnp.transpose(weights[1])[None, :, None, :]).reshape(
                  8 * l, 8 * dout)
    bias_wide = jnp.tile(bias, (1, 8))
    hs = _tc_matmul(g0, g1, perm_vals, pool_vals.reshape(p, 1),
                    w0_big, w1_big, bias_wide, p, l, dout)

    n_win = -(-n_nodes // _WIN)
    cuts = jnp.arange(1, n_win + 1, dtype=jnp.int32) * _WIN
    bounds = jnp.searchsorted(pool_seg_ids, cuts, side="left").astype(jnp.int32)
    bounds = jnp.concatenate([jnp.zeros((1,), jnp.int32), bounds])
    bounds = jnp.pad(bounds, (0, 16 - bounds.shape[0]),
                     constant_values=jnp.int32(m // l))
    pooled = _sc_segsum(hs, pool_seg_ids, bounds, n_nodes)

    out = _tc_finish(pooled, degs[:, None], jnp.transpose(W0), b0[None, :],
                     jnp.transpose(W1), b1[None, :])
    return out


# speedup vs baseline: 36.8490x; 1.2203x over previous
import functools

import jax
import jax.numpy as jnp
from jax import lax
from jax.experimental import pallas as pl
from jax.experimental.pallas import tpu as pltpu
from jax.experimental.pallas import tpu_sc as plsc

_NCORES = 2
_NSUB = 16
_WIN = 8192
_CHUNK = 128


def _sc_gather(nfeat_packed, perm_idx, perm_vals):
    n = nfeat_packed.shape[0]
    m = perm_idx.shape[0]
    per_tile = m // (_NCORES * _NSUB)
    ch = 2000
    n_ch = per_tile // ch
    assert per_tile % ch == 0 and n_ch % 2 == 0

    mesh = plsc.VectorSubcoreMesh(core_axis_name="c", subcore_axis_name="s")

    @functools.partial(
        pl.kernel,
        mesh=mesh,
        out_type=jax.ShapeDtypeStruct((m,), jnp.int32),
        scratch_types=[pltpu.VMEM((n,), jnp.int32),
                       pltpu.VMEM((ch,), jnp.int32),
                       pltpu.VMEM((ch,), jnp.int32),
                       pltpu.VMEM((ch,), jnp.float32),
                       pltpu.VMEM((ch,), jnp.float32),
                       pltpu.VMEM((ch,), jnp.int32),
                       pltpu.VMEM((ch,), jnp.int32),
                       pltpu.SemaphoreType.DMA,
                       pltpu.SemaphoreType.DMA,
                       pltpu.SemaphoreType.DMA,
                       pltpu.SemaphoreType.DMA],
        compiler_params=pltpu.CompilerParams(needs_layout_passes=False),
    )
    def k(tab_hbm, idx_hbm, vals_hbm, xi_hbm, table_v, ibuf0, ibuf1,
          vbuf0, vbuf1, obuf0, obuf1, sem_in0, sem_in1, sem_out0, sem_out1):
        c = lax.axis_index("c")
        s = lax.axis_index("s")
        wid = s * _NCORES + c
        base = wid * per_tile
        ibufs = (ibuf0, ibuf1)
        vbufs = (vbuf0, vbuf1)
        obufs = (obuf0, obuf1)
        sems_in = (sem_in0, sem_in1)
        sems_out = (sem_out0, sem_out1)

        def start_in(j, b):
            off = base + j * ch
            pltpu.make_async_copy(idx_hbm.at[pl.ds(off, ch)], ibufs[b],
                                  sems_in[b]).start()
            pltpu.make_async_copy(vals_hbm.at[pl.ds(off, ch)], vbufs[b],
                                  sems_in[b]).start()

        def wait_in(j, b):
            off = base + j * ch
            pltpu.make_async_copy(idx_hbm.at[pl.ds(off, ch)], ibufs[b],
                                  sems_in[b]).wait()
            pltpu.make_async_copy(vals_hbm.at[pl.ds(off, ch)], vbufs[b],
                                  sems_in[b]).wait()

        def out_desc(j, b):
            off = base + j * ch
            return pltpu.make_async_copy(obufs[b],
                                         xi_hbm.at[pl.ds(off, ch)],
                                         sems_out[b])

        pltpu.sync_copy(tab_hbm, table_v)
        start_in(0, 0)
        start_in(1, 1)

        def pair(t, carry):
            for b in range(2):
                j = 2 * t + b
                wait_in(j, b)

                @pl.when(j >= 2)
                def _(j=j, b=b):
                    out_desc(j - 2, b).wait()

                def gbody(i, carry2, b=b):
                    idxv = ibufs[b][pl.ds(i * 16, 16)]
                    pk = plsc.load_gather(table_v, [idxv])
                    xb = plsc.bitcast(pk, jnp.bfloat16)
                    vv = vbufs[b][pl.ds(i * 16, 16)]
                    vp = plsc.pack(vv, vv, format=plsc.PackFormat.INTERLEAVED)
                    obufs[b][pl.ds(i * 16, 16)] = plsc.bitcast(
                        xb * vp, jnp.int32)
                    return carry2

                lax.fori_loop(0, ch // 16, gbody, 0, unroll=8)

                @pl.when(j + 2 < n_ch)
                def _(j=j, b=b):
                    start_in(j + 2, b)

                out_desc(j, b).start()
            return carry

        lax.fori_loop(0, n_ch // 2, pair, 0)
        out_desc(n_ch - 2, 0).wait()
        out_desc(n_ch - 1, 1).wait()

    return k(nfeat_packed, perm_idx, perm_vals)


def _tc_matmul(xi, pool_vals, w0_big, w1_big, bias_wide, p, l, dout):
    bp = 3200
    assert p % bp == 0
    rows = bp * l // 128

    def body(x_r, pv_r, w0_r, w1_r, b_r, o_r):
        xw = x_r[...]
        x0 = lax.bitcast_convert_type(xw << 16, jnp.float32)
        x1 = lax.bitcast_convert_type(xw & jnp.int32(-65536), jnp.float32)
        hw = jnp.dot(x0.astype(jnp.bfloat16), w0_r[...],
                     preferred_element_type=jnp.float32)
        hw += jnp.dot(x1.astype(jnp.bfloat16), w1_r[...],
                      preferred_element_type=jnp.float32)
        hw = jnp.maximum(hw + b_r[...], 0.0)
        o_r[...] = hw.reshape(bp, dout) * pv_r[...]

    xiv = xi.reshape(p * l // 128, 128)
    return pl.pallas_call(
        body,
        grid=(p // bp,),
        in_specs=[pl.BlockSpec((rows, 128), lambda i: (i, 0)),
                  pl.BlockSpec((bp, 1), lambda i: (i, 0)),
                  pl.BlockSpec((128, 8 * dout), lambda i: (0, 0)),
                  pl.BlockSpec((128, 8 * dout), lambda i: (0, 0)),
                  pl.BlockSpec((1, 8 * dout), lambda i: (0, 0))],
        out_specs=pl.BlockSpec((bp, dout), lambda i: (i, 0)),
        out_shape=jax.ShapeDtypeStruct((p, dout), jnp.float32),
    )(xiv, pool_vals, w0_big, w1_big, bias_wide)


def _sc_segsum(hs, seg, bounds, n_nodes):
    p, dout = hs.shape
    n_win = -(-n_nodes // _WIN)
    zrows = 520
    wbuf_rows = zrows * _NSUB
    assert p % _CHUNK == 0

    mesh = plsc.VectorSubcoreMesh(core_axis_name="c", subcore_axis_name="s")

    @functools.partial(
        pl.kernel,
        mesh=mesh,
        out_type=jax.ShapeDtypeStruct((n_nodes, dout), jnp.float32),
        scratch_types=[pltpu.VMEM((_CHUNK, dout), jnp.float32),
                       pltpu.VMEM((_CHUNK,), jnp.int32),
                       pltpu.VMEM((_CHUNK,), jnp.int32),
                       pltpu.VMEM((16,), jnp.int32),
                       pltpu.VMEM((_CHUNK, dout), jnp.float32),
                       pltpu.VMEM_SHARED((wbuf_rows, dout), jnp.float32)],
        compiler_params=pltpu.CompilerParams(needs_layout_passes=False),
    )
    def k(hs_hbm, seg_hbm, bounds_hbm, out_hbm,
          rows_v, seg_v, idx_v, bounds_v, zbuf_v, win_sh):
        c = lax.axis_index("c")
        s = lax.axis_index("s")
        pltpu.sync_copy(bounds_hbm, bounds_v)
        bvec = bounds_v[...]
        iot = lax.iota(jnp.int32, 16)

        def zb(i, carry):
            for u in range(dout // 16):
                zbuf_v[i, pl.ds(u * 16, 16)] = jnp.zeros((16,), jnp.float32)
            return carry

        lax.fori_loop(0, _CHUNK, zb, 0)

        def extract(idx_static):
            return jnp.max(jnp.where(iot == idx_static, bvec,
                                     jnp.int32(-2147483648)))

        for w in range(n_win):
            n0 = w * _WIN
            wn = min(_WIN, n_nodes - n0)

            @pl.when(c == (w % _NCORES))
            def _(w=w, n0=n0, wn=wn):
                r0 = extract(w)
                r1 = extract(w + 1)
                nfull, ztail = divmod(zrows, _CHUNK)
                for q in range(nfull):
                    pltpu.sync_copy(
                        zbuf_v,
                        win_sh.at[pl.ds(s * zrows + q * _CHUNK, _CHUNK)])
                if ztail:
                    pltpu.sync_copy(
                        zbuf_v.at[pl.ds(0, ztail)],
                        win_sh.at[pl.ds(s * zrows + nfull * _CHUNK, ztail)])
                plsc.subcore_barrier()

                k0 = r0 // _CHUNK
                k1 = (r1 + _CHUNK - 1) // _CHUNK
                nk = jnp.maximum(0, (k1 - k0 - s + _NSUB - 1) // _NSUB)

                def chunk(t, carry):
                    kk = k0 + s + t * _NSUB
                    rbase = kk * _CHUNK
                    pltpu.sync_copy(hs_hbm.at[pl.ds(rbase, _CHUNK)], rows_v)
                    pltpu.sync_copy(seg_hbm.at[pl.ds(rbase, _CHUNK)], seg_v)
                    for u in range(_CHUNK // 16):
                        sv = seg_v[pl.ds(u * 16, 16)]
                        absrow = rbase + u * 16 + iot
                        valid = (absrow >= r0) & (absrow < r1)
                        li = jnp.where(valid, sv - n0,
                                       jnp.int32(_WIN) + s)
                        idx_v[pl.ds(u * 16, 16)] = li
                    pltpu.sync_copy(rows_v, win_sh.at[idx_v], add=True)
                    return carry

                lax.fori_loop(0, nk, chunk, 0)
                plsc.subcore_barrier()

                share = (wn // _NSUB) & ~7
                if share > 0:
                    pltpu.sync_copy(
                        win_sh.at[pl.ds(s * share, share)],
                        out_hbm.at[pl.ds(n0 + s * share, share)])
                rem = wn - share * _NSUB
                if rem > 0:
                    @pl.when(s == 0)
                    def _(share=share, rem=rem, n0=n0):
                        pltpu.sync_copy(
                            win_sh.at[pl.ds(share * _NSUB, rem)],
                            out_hbm.at[pl.ds(n0 + share * _NSUB, rem)])
                plsc.subcore_barrier()

    return k(hs, seg, bounds)


def _tc_finish(pooled, degs2d, w0t, b0, w1t, b1):
    n, dout = pooled.shape
    dh = w0t.shape[1]
    bn = 2000
    assert n % bn == 0

    def body(p_r, d_r, w0_r, b0_r, w1_r, b1_r, o_r):
        f1 = jnp.maximum(d_r[...] * w0_r[...] + b0_r[...], 0.0)
        f = jnp.dot(f1, w1_r[...], preferred_element_type=jnp.float32)
        f = f + b1_r[...]
        o_r[...] = jnp.maximum(p_r[...] * f, 0.0)

    return pl.pallas_call(
        body,
        grid=(n // bn,),
        in_specs=[pl.BlockSpec((bn, dout), lambda i: (i, 0)),
                  pl.BlockSpec((bn, 1), lambda i: (i, 0)),
                  pl.BlockSpec((1, dh), lambda i: (0, 0)),
                  pl.BlockSpec((1, dh), lambda i: (0, 0)),
                  pl.BlockSpec((dh, dout), lambda i: (0, 0)),
                  pl.BlockSpec((1, dout), lambda i: (0, 0))],
        out_specs=pl.BlockSpec((bn, dout), lambda i: (i, 0)),
        out_shape=jax.ShapeDtypeStruct((n, dout), jnp.float32),
    )(pooled, degs2d, w0t, b0, w1t, b1)


def kernel(nfeat, perm_idx, perm_vals, pool_seg_ids, pool_vals, degs,
           weights, bias, W0, b0, W1, b1):
    n_nodes, d_in = nfeat.shape
    l = weights.shape[2]
    dout = weights.shape[1]
    m = perm_idx.shape[0]
    p = m // l

    packed = lax.bitcast_convert_type(nfeat.astype(jnp.bfloat16), jnp.int32)
    xi = _sc_gather(packed, perm_idx, perm_vals)

    eye8 = jnp.eye(8, dtype=jnp.float32)
    w0_big = (eye8[:, None, :, None]
              * jnp.transpose(weights[0])[None, :, None, :]).reshape(
                  8 * l, 8 * dout).astype(jnp.bfloat16)
    w1_big = (eye8[:, None, :, None]
              * jnp.transpose(weights[1])[None, :, None, :]).reshape(
                  8 * l, 8 * dout).astype(jnp.bfloat16)
    bias_wide = jnp.tile(bias, (1, 8))
    hs = _tc_matmul(xi, pool_vals.reshape(p, 1),
                    w0_big, w1_big, bias_wide, p, l, dout)

    n_win = -(-n_nodes // _WIN)
    cuts = jnp.arange(1, n_win + 1, dtype=jnp.int32) * _WIN
    bounds = jnp.searchsorted(pool_seg_ids, cuts, side="left").astype(jnp.int32)
    bounds = jnp.concatenate([jnp.zeros((1,), jnp.int32), bounds])
    bounds = jnp.pad(bounds, (0, 16 - bounds.shape[0]),
                     constant_values=jnp.int32(m // l))
    pooled = _sc_segsum(hs, pool_seg_ids, bounds, n_nodes)

    out = _tc_finish(pooled, degs[:, None], jnp.transpose(W0), b0[None, :],
                     jnp.transpose(W1), b1[None, :])
    return out


# speedup vs baseline: 42.0804x; 1.1420x over previous
import functools

import jax
import jax.numpy as jnp
from jax import lax
from jax.experimental import pallas as pl
from jax.experimental.pallas import tpu as pltpu
from jax.experimental.pallas import tpu_sc as plsc

_NCORES = 2
_NSUB = 16
_WIN = 8192
_CHUNK = 128


def _sc_gather(nfeat_packed, perm_idx, perm_vals):
    n = nfeat_packed.shape[0]
    m = perm_idx.shape[0]
    per_tile = m // (_NCORES * _NSUB)
    ch = 2000
    n_ch = per_tile // ch
    assert per_tile % ch == 0 and n_ch % 2 == 0

    mesh = plsc.VectorSubcoreMesh(core_axis_name="c", subcore_axis_name="s")

    @functools.partial(
        pl.kernel,
        mesh=mesh,
        out_type=jax.ShapeDtypeStruct((m,), jnp.int32),
        scratch_types=[pltpu.VMEM((n,), jnp.int32),
                       pltpu.VMEM((ch,), jnp.int32),
                       pltpu.VMEM((ch,), jnp.int32),
                       pltpu.VMEM((ch,), jnp.float32),
                       pltpu.VMEM((ch,), jnp.float32),
                       pltpu.VMEM((ch,), jnp.int32),
                       pltpu.VMEM((ch,), jnp.int32),
                       pltpu.SemaphoreType.DMA,
                       pltpu.SemaphoreType.DMA,
                       pltpu.SemaphoreType.DMA,
                       pltpu.SemaphoreType.DMA],
        compiler_params=pltpu.CompilerParams(needs_layout_passes=False),
    )
    def k(tab_hbm, idx_hbm, vals_hbm, xi_hbm, table_v, ibuf0, ibuf1,
          vbuf0, vbuf1, obuf0, obuf1, sem_in0, sem_in1, sem_out0, sem_out1):
        c = lax.axis_index("c")
        s = lax.axis_index("s")
        wid = s * _NCORES + c
        base = wid * per_tile
        ibufs = (ibuf0, ibuf1)
        vbufs = (vbuf0, vbuf1)
        obufs = (obuf0, obuf1)
        sems_in = (sem_in0, sem_in1)
        sems_out = (sem_out0, sem_out1)

        def start_in(j, b):
            off = base + j * ch
            pltpu.make_async_copy(idx_hbm.at[pl.ds(off, ch)], ibufs[b],
                                  sems_in[b]).start()
            pltpu.make_async_copy(vals_hbm.at[pl.ds(off, ch)], vbufs[b],
                                  sems_in[b]).start()

        def wait_in(j, b):
            off = base + j * ch
            pltpu.make_async_copy(idx_hbm.at[pl.ds(off, ch)], ibufs[b],
                                  sems_in[b]).wait()
            pltpu.make_async_copy(vals_hbm.at[pl.ds(off, ch)], vbufs[b],
                                  sems_in[b]).wait()

        def out_desc(j, b):
            off = base + j * ch
            return pltpu.make_async_copy(obufs[b],
                                         xi_hbm.at[pl.ds(off, ch)],
                                         sems_out[b])

        pltpu.sync_copy(tab_hbm, table_v)
        start_in(0, 0)
        start_in(1, 1)

        def pair(t, carry):
            for b in range(2):
                j = 2 * t + b
                wait_in(j, b)

                @pl.when(j >= 2)
                def _(j=j, b=b):
                    out_desc(j - 2, b).wait()

                def gbody(i, carry2, b=b):
                    idxv = ibufs[b][pl.ds(i * 16, 16)]
                    pk = plsc.load_gather(table_v, [idxv])
                    xb = plsc.bitcast(pk, jnp.bfloat16)
                    vv = vbufs[b][pl.ds(i * 16, 16)]
                    vp = plsc.pack(vv, vv, format=plsc.PackFormat.INTERLEAVED)
                    obufs[b][pl.ds(i * 16, 16)] = plsc.bitcast(
                        xb * vp, jnp.int32)
                    return carry2

                lax.fori_loop(0, ch // 16, gbody, 0, unroll=8)

                @pl.when(j + 2 < n_ch)
                def _(j=j, b=b):
                    start_in(j + 2, b)

                out_desc(j, b).start()
            return carry

        lax.fori_loop(0, n_ch // 2, pair, 0)
        out_desc(n_ch - 2, 0).wait()
        out_desc(n_ch - 1, 1).wait()

    return k(nfeat_packed, perm_idx, perm_vals)


def _tc_matmul(xi, pool_vals, w0_big, w1_big, bias_wide, p, l, dout):
    bp = 3200
    assert p % bp == 0
    rows = bp * l // 128

    def body(x_r, pv_r, w0_r, w1_r, b_r, o_r):
        xw = x_r[...]
        x0 = lax.bitcast_convert_type(xw << 16, jnp.float32)
        x1 = lax.bitcast_convert_type(xw & jnp.int32(-65536), jnp.float32)
        hw = jnp.dot(x0.astype(jnp.bfloat16), w0_r[...],
                     preferred_element_type=jnp.float32)
        hw += jnp.dot(x1.astype(jnp.bfloat16), w1_r[...],
                      preferred_element_type=jnp.float32)
        hw = jnp.maximum(hw + b_r[...], 0.0)
        o_r[...] = hw.reshape(bp, dout) * pv_r[...]

    xiv = xi.reshape(p * l // 128, 128)
    return pl.pallas_call(
        body,
        grid=(p // bp,),
        in_specs=[pl.BlockSpec((rows, 128), lambda i: (i, 0)),
                  pl.BlockSpec((bp, 1), lambda i: (i, 0)),
                  pl.BlockSpec((128, 8 * dout), lambda i: (0, 0)),
                  pl.BlockSpec((128, 8 * dout), lambda i: (0, 0)),
                  pl.BlockSpec((1, 8 * dout), lambda i: (0, 0))],
        out_specs=pl.BlockSpec((bp, dout), lambda i: (i, 0)),
        out_shape=jax.ShapeDtypeStruct((p, dout), jnp.float32),
    )(xiv, pool_vals, w0_big, w1_big, bias_wide)


def _sc_segsum(hs, seg, bounds, n_nodes):
    p, dout = hs.shape
    n_win = -(-n_nodes // _WIN)
    zrows = 528
    wbuf_rows = zrows * _NSUB
    assert p % _CHUNK == 0

    mesh = plsc.VectorSubcoreMesh(core_axis_name="c", subcore_axis_name="s")

    @functools.partial(
        pl.kernel,
        mesh=mesh,
        out_type=jax.ShapeDtypeStruct((n_nodes, dout), jnp.float32),
        scratch_types=[pltpu.VMEM((_CHUNK, dout), jnp.float32),
                       pltpu.VMEM((_CHUNK, dout), jnp.float32),
                       pltpu.VMEM((_CHUNK,), jnp.int32),
                       pltpu.VMEM((_CHUNK,), jnp.int32),
                       pltpu.VMEM((_CHUNK,), jnp.int32),
                       pltpu.VMEM((_CHUNK,), jnp.int32),
                       pltpu.VMEM((16,), jnp.int32),
                       pltpu.VMEM((_CHUNK, dout), jnp.float32),
                       pltpu.VMEM_SHARED((wbuf_rows, dout), jnp.float32),
                       pltpu.SemaphoreType.DMA,
                       pltpu.SemaphoreType.DMA],
        compiler_params=pltpu.CompilerParams(needs_layout_passes=False),
    )
    def k(hs_hbm, seg_hbm, bounds_hbm, out_hbm,
          rows0, rows1, seg0, seg1, idx0, idx1, bounds_v, zbuf_v, win_sh,
          sem0, sem1):
        c = lax.axis_index("c")
        s = lax.axis_index("s")
        rowsb = (rows0, rows1)
        segb = (seg0, seg1)
        idxb = (idx0, idx1)
        sems = (sem0, sem1)
        pltpu.sync_copy(bounds_hbm, bounds_v)
        bvec = bounds_v[...]
        iot = lax.iota(jnp.int32, 16)

        def in_start(kk, b):
            rbase = kk * _CHUNK
            pltpu.make_async_copy(hs_hbm.at[pl.ds(rbase, _CHUNK)],
                                  rowsb[b], sems[b]).start()
            pltpu.make_async_copy(seg_hbm.at[pl.ds(rbase, _CHUNK)],
                                  segb[b], sems[b]).start()

        def in_wait(kk, b):
            rbase = kk * _CHUNK
            pltpu.make_async_copy(hs_hbm.at[pl.ds(rbase, _CHUNK)],
                                  rowsb[b], sems[b]).wait()
            pltpu.make_async_copy(seg_hbm.at[pl.ds(rbase, _CHUNK)],
                                  segb[b], sems[b]).wait()

        def process(kk, b, r0, r1, n0):
            rbase = kk * _CHUNK
            for u in range(_CHUNK // 16):
                sv = segb[b][pl.ds(u * 16, 16)]
                absrow = rbase + u * 16 + iot
                valid = (absrow >= r0) & (absrow < r1)
                li = jnp.where(valid, sv - n0, jnp.int32(_WIN) + s)
                idxb[b][pl.ds(u * 16, 16)] = li
            pltpu.sync_copy(rowsb[b], win_sh.at[idxb[b]], add=True)

        def zb(i, carry):
            for u in range(dout // 16):
                zbuf_v[i, pl.ds(u * 16, 16)] = jnp.zeros((16,), jnp.float32)
            return carry

        lax.fori_loop(0, _CHUNK, zb, 0)

        def extract(idx_static):
            return jnp.max(jnp.where(iot == idx_static, bvec,
                                     jnp.int32(-2147483648)))

        for w in range(n_win):
            n0 = w * _WIN
            wn = min(_WIN, n_nodes - n0)

            @pl.when(c == (w % _NCORES))
            def _(w=w, n0=n0, wn=wn):
                r0 = extract(w)
                r1 = extract(w + 1)
                nfull, ztail = divmod(zrows, _CHUNK)
                for q in range(nfull):
                    pltpu.sync_copy(
                        zbuf_v,
                        win_sh.at[pl.ds(s * zrows + q * _CHUNK, _CHUNK)])
                if ztail:
                    pltpu.sync_copy(
                        zbuf_v.at[pl.ds(0, ztail)],
                        win_sh.at[pl.ds(s * zrows + nfull * _CHUNK, ztail)])
                plsc.subcore_barrier()

                k0 = r0 // _CHUNK
                k1 = (r1 + _CHUNK - 1) // _CHUNK
                nk = jnp.maximum(0, (k1 - k0 - s + _NSUB - 1) // _NSUB)
                odd = nk & 1

                @pl.when(odd == 1)
                def _(r0=r0, r1=r1, n0=n0):
                    kk = k0 + s
                    in_start(kk, 0)
                    in_wait(kk, 0)
                    process(kk, 0, r0, r1, n0)

                cb = k0 + s + odd * _NSUB
                rem = nk - odd

                @pl.when(rem > 0)
                def _(cb=cb):
                    in_start(cb, 0)

                def pair(t, carry, r0=r0, r1=r1, n0=n0, cb=cb, rem=rem):
                    for b in range(2):
                        kk = cb + (2 * t + b) * _NSUB
                        in_wait(kk, b)
                        nxt = 2 * t + b + 1

                        @pl.when(nxt < rem)
                        def _(nxt=nxt, b=b):
                            in_start(cb + nxt * _NSUB, 1 - b)

                        process(kk, b, r0, r1, n0)
                    return carry

                lax.fori_loop(0, rem >> 1, pair, 0)
                plsc.subcore_barrier()

                share = (wn // _NSUB) & ~7
                if share > 0:
                    pltpu.sync_copy(
                        win_sh.at[pl.ds(s * share, share)],
                        out_hbm.at[pl.ds(n0 + s * share, share)])
                rem = wn - share * _NSUB
                if rem > 0:
                    @pl.when(s == 0)
                    def _(share=share, rem=rem, n0=n0):
                        pltpu.sync_copy(
                            win_sh.at[pl.ds(share * _NSUB, rem)],
                            out_hbm.at[pl.ds(n0 + share * _NSUB, rem)])
                plsc.subcore_barrier()

    return k(hs, seg, bounds)


def _tc_finish(pooled, degs2d, w0t, b0, w1t, b1):
    n, dout = pooled.shape
    dh = w0t.shape[1]
    bn = 2000
    assert n % bn == 0

    def body(p_r, d_r, w0_r, b0_r, w1_r, b1_r, o_r):
        f1 = jnp.maximum(d_r[...] * w0_r[...] + b0_r[...], 0.0)
        f = jnp.dot(f1, w1_r[...], preferred_element_type=jnp.float32)
        f = f + b1_r[...]
        o_r[...] = jnp.maximum(p_r[...] * f, 0.0)

    return pl.pallas_call(
        body,
        grid=(n // bn,),
        in_specs=[pl.BlockSpec((bn, dout), lambda i: (i, 0)),
                  pl.BlockSpec((bn, 1), lambda i: (i, 0)),
                  pl.BlockSpec((1, dh), lambda i: (0, 0)),
                  pl.BlockSpec((1, dh), lambda i: (0, 0)),
                  pl.BlockSpec((dh, dout), lambda i: (0, 0)),
                  pl.BlockSpec((1, dout), lambda i: (0, 0))],
        out_specs=pl.BlockSpec((bn, dout), lambda i: (i, 0)),
        out_shape=jax.ShapeDtypeStruct((n, dout), jnp.float32),
    )(pooled, degs2d, w0t, b0, w1t, b1)


def kernel(nfeat, perm_idx, perm_vals, pool_seg_ids, pool_vals, degs,
           weights, bias, W0, b0, W1, b1):
    n_nodes, d_in = nfeat.shape
    l = weights.shape[2]
    dout = weights.shape[1]
    m = perm_idx.shape[0]
    p = m // l

    packed = lax.bitcast_convert_type(nfeat.astype(jnp.bfloat16), jnp.int32)
    xi = _sc_gather(packed, perm_idx, perm_vals)

    eye8 = jnp.eye(8, dtype=jnp.float32)
    w0_big = (eye8[:, None, :, None]
              * jnp.transpose(weights[0])[None, :, None, :]).reshape(
                  8 * l, 8 * dout).astype(jnp.bfloat16)
    w1_big = (eye8[:, None, :, None]
              * jnp.transpose(weights[1])[None, :, None, :]).reshape(
                  8 * l, 8 * dout).astype(jnp.bfloat16)
    bias_wide = jnp.tile(bias, (1, 8))
    hs = _tc_matmul(xi, pool_vals.reshape(p, 1),
                    w0_big, w1_big, bias_wide, p, l, dout)

    n_win = -(-n_nodes // _WIN)
    cuts = jnp.arange(1, n_win + 1, dtype=jnp.int32) * _WIN
    bounds = jnp.searchsorted(pool_seg_ids, cuts, side="left").astype(jnp.int32)
    bounds = jnp.concatenate([jnp.zeros((1,), jnp.int32), bounds])
    bounds = jnp.pad(bounds, (0, 16 - bounds.shape[0]),
                     constant_values=jnp.int32(m // l))
    pooled = _sc_segsum(hs, pool_seg_ids, bounds, n_nodes)

    out = _tc_finish(pooled, degs[:, None], jnp.transpose(W0), b0[None, :],
                     jnp.transpose(W1), b1[None, :])
    return out


# speedup vs baseline: 42.1173x; 1.0009x over previous
import functools

import jax
import jax.numpy as jnp
from jax import lax
from jax.experimental import pallas as pl
from jax.experimental.pallas import tpu as pltpu
from jax.experimental.pallas import tpu_sc as plsc

_NCORES = 2
_NSUB = 16
_WIN = 8192
_CHUNK = 128


def _sc_gather(nfeat_packed, perm_idx, perm_vals):
    n = nfeat_packed.shape[0]
    m = perm_idx.shape[0]
    per_tile = m // (_NCORES * _NSUB)
    ch = 2000
    n_ch = per_tile // ch
    assert per_tile % ch == 0 and n_ch % 2 == 0

    mesh = plsc.VectorSubcoreMesh(core_axis_name="c", subcore_axis_name="s")

    @functools.partial(
        pl.kernel,
        mesh=mesh,
        out_type=jax.ShapeDtypeStruct((m,), jnp.int32),
        scratch_types=[pltpu.VMEM((n,), jnp.int32),
                       pltpu.VMEM((ch,), jnp.int32),
                       pltpu.VMEM((ch,), jnp.int32),
                       pltpu.VMEM((ch,), jnp.float32),
                       pltpu.VMEM((ch,), jnp.float32),
                       pltpu.VMEM((ch,), jnp.int32),
                       pltpu.VMEM((ch,), jnp.int32),
                       pltpu.SemaphoreType.DMA,
                       pltpu.SemaphoreType.DMA,
                       pltpu.SemaphoreType.DMA,
                       pltpu.SemaphoreType.DMA],
        compiler_params=pltpu.CompilerParams(needs_layout_passes=False),
    )
    def k(tab_hbm, idx_hbm, vals_hbm, xi_hbm, table_v, ibuf0, ibuf1,
          vbuf0, vbuf1, obuf0, obuf1, sem_in0, sem_in1, sem_out0, sem_out1):
        c = lax.axis_index("c")
        s = lax.axis_index("s")
        wid = s * _NCORES + c
        base = wid * per_tile
        ibufs = (ibuf0, ibuf1)
        vbufs = (vbuf0, vbuf1)
        obufs = (obuf0, obuf1)
        sems_in = (sem_in0, sem_in1)
        sems_out = (sem_out0, sem_out1)

        def start_in(j, b):
            off = base + j * ch
            pltpu.make_async_copy(idx_hbm.at[pl.ds(off, ch)], ibufs[b],
                                  sems_in[b]).start()
            pltpu.make_async_copy(vals_hbm.at[pl.ds(off, ch)], vbufs[b],
                                  sems_in[b]).start()

        def wait_in(j, b):
            off = base + j * ch
            pltpu.make_async_copy(idx_hbm.at[pl.ds(off, ch)], ibufs[b],
                                  sems_in[b]).wait()
            pltpu.make_async_copy(vals_hbm.at[pl.ds(off, ch)], vbufs[b],
                                  sems_in[b]).wait()

        def out_desc(j, b):
            off = base + j * ch
            return pltpu.make_async_copy(obufs[b],
                                         xi_hbm.at[pl.ds(off, ch)],
                                         sems_out[b])

        pltpu.sync_copy(tab_hbm, table_v)
        start_in(0, 0)
        start_in(1, 1)

        def pair(t, carry):
            for b in range(2):
                j = 2 * t + b
                wait_in(j, b)

                @pl.when(j >= 2)
                def _(j=j, b=b):
                    out_desc(j - 2, b).wait()

                def gbody(i, carry2, b=b):
                    idxv = ibufs[b][pl.ds(i * 16, 16)]
                    pk = plsc.load_gather(table_v, [idxv])
                    xb = plsc.bitcast(pk, jnp.bfloat16)
                    vv = vbufs[b][pl.ds(i * 16, 16)]
                    vp = plsc.pack(vv, vv, format=plsc.PackFormat.INTERLEAVED)
                    obufs[b][pl.ds(i * 16, 16)] = plsc.bitcast(
                        xb * vp, jnp.int32)
                    return carry2

                lax.fori_loop(0, ch // 16, gbody, 0, unroll=8)

                @pl.when(j + 2 < n_ch)
                def _(j=j, b=b):
                    start_in(j + 2, b)

                out_desc(j, b).start()
            return carry

        lax.fori_loop(0, n_ch // 2, pair, 0)
        out_desc(n_ch - 2, 0).wait()
        out_desc(n_ch - 1, 1).wait()

    return k(nfeat_packed, perm_idx, perm_vals)


def _tc_matmul(xi, pool_vals, w0_big, w1_big, bias_wide, p, l, dout):
    bp = 3200
    assert p % bp == 0
    rows = bp * l // 128

    def body(x_r, pv_r, w0_r, w1_r, b_r, o_r):
        xw = x_r[...]
        x0 = lax.bitcast_convert_type(xw << 16, jnp.float32)
        x1 = lax.bitcast_convert_type(xw & jnp.int32(-65536), jnp.float32)
        hw = jnp.dot(x0.astype(jnp.bfloat16), w0_r[...],
                     preferred_element_type=jnp.float32)
        hw += jnp.dot(x1.astype(jnp.bfloat16), w1_r[...],
                      preferred_element_type=jnp.float32)
        hw = jnp.maximum(hw + b_r[...], 0.0)
        o_r[...] = hw.reshape(bp, dout) * pv_r[...]

    xiv = xi.reshape(p * l // 128, 128)
    return pl.pallas_call(
        body,
        grid=(p // bp,),
        in_specs=[pl.BlockSpec((rows, 128), lambda i: (i, 0)),
                  pl.BlockSpec((bp, 1), lambda i: (i, 0)),
                  pl.BlockSpec((128, 8 * dout), lambda i: (0, 0)),
                  pl.BlockSpec((128, 8 * dout), lambda i: (0, 0)),
                  pl.BlockSpec((1, 8 * dout), lambda i: (0, 0))],
        out_specs=pl.BlockSpec((bp, dout), lambda i: (i, 0)),
        out_shape=jax.ShapeDtypeStruct((p, dout), jnp.float32),
    )(xiv, pool_vals, w0_big, w1_big, bias_wide)


def _sc_segsum(hs, seg, bounds, n_nodes):
    p, dout = hs.shape
    n_win = -(-n_nodes // _WIN)
    zrows = 528
    wbuf_rows = zrows * _NSUB
    assert p % _CHUNK == 0

    mesh = plsc.VectorSubcoreMesh(core_axis_name="c", subcore_axis_name="s")

    @functools.partial(
        pl.kernel,
        mesh=mesh,
        out_type=jax.ShapeDtypeStruct((n_nodes, dout), jnp.float32),
        scratch_types=[pltpu.VMEM((_CHUNK, dout), jnp.float32),
                       pltpu.VMEM((_CHUNK, dout), jnp.float32),
                       pltpu.VMEM((_CHUNK,), jnp.int32),
                       pltpu.VMEM((_CHUNK,), jnp.int32),
                       pltpu.VMEM((_CHUNK,), jnp.int32),
                       pltpu.VMEM((_CHUNK,), jnp.int32),
                       pltpu.VMEM((16,), jnp.int32),
                       pltpu.VMEM((_CHUNK, dout), jnp.float32),
                       pltpu.VMEM_SHARED((wbuf_rows, dout), jnp.float32),
                       pltpu.SemaphoreType.DMA,
                       pltpu.SemaphoreType.DMA,
                       pltpu.SemaphoreType.DMA,
                       pltpu.SemaphoreType.DMA],
        compiler_params=pltpu.CompilerParams(needs_layout_passes=False),
    )
    def k(hs_hbm, seg_hbm, bounds_hbm, out_hbm,
          rows0, rows1, seg0, seg1, idx0, idx1, bounds_v, zbuf_v, win_sh,
          sem0, sem1, sem_sc0, sem_sc1):
        c = lax.axis_index("c")
        s = lax.axis_index("s")
        rowsb = (rows0, rows1)
        segb = (seg0, seg1)
        idxb = (idx0, idx1)
        sems = (sem0, sem1)
        sems_sc = (sem_sc0, sem_sc1)
        pltpu.sync_copy(bounds_hbm, bounds_v)
        bvec = bounds_v[...]
        iot = lax.iota(jnp.int32, 16)

        def in_start(kk, b):
            rbase = kk * _CHUNK
            pltpu.make_async_copy(hs_hbm.at[pl.ds(rbase, _CHUNK)],
                                  rowsb[b], sems[b]).start()
            pltpu.make_async_copy(seg_hbm.at[pl.ds(rbase, _CHUNK)],
                                  segb[b], sems[b]).start()

        def in_wait(kk, b):
            rbase = kk * _CHUNK
            pltpu.make_async_copy(hs_hbm.at[pl.ds(rbase, _CHUNK)],
                                  rowsb[b], sems[b]).wait()
            pltpu.make_async_copy(seg_hbm.at[pl.ds(rbase, _CHUNK)],
                                  segb[b], sems[b]).wait()

        def compute_idx(kk, b, r0, r1, n0):
            rbase = kk * _CHUNK
            for u in range(_CHUNK // 16):
                sv = segb[b][pl.ds(u * 16, 16)]
                absrow = rbase + u * 16 + iot
                valid = (absrow >= r0) & (absrow < r1)
                li = jnp.where(valid, sv - n0, jnp.int32(_WIN) + s)
                idxb[b][pl.ds(u * 16, 16)] = li

        def sc_start(b):
            pltpu.async_copy(rowsb[b], win_sh.at[idxb[b]], sems_sc[b],
                             add=True)

        def sc_wait(b):
            pltpu.make_async_copy(rowsb[b], win_sh.at[idxb[b]],
                                  sems_sc[b]).wait()

        def process(kk, b, r0, r1, n0):
            compute_idx(kk, b, r0, r1, n0)
            pltpu.sync_copy(rowsb[b], win_sh.at[idxb[b]], add=True)

        def zb(i, carry):
            for u in range(dout // 16):
                zbuf_v[i, pl.ds(u * 16, 16)] = jnp.zeros((16,), jnp.float32)
            return carry

        lax.fori_loop(0, _CHUNK, zb, 0)

        def extract(idx_static):
            return jnp.max(jnp.where(iot == idx_static, bvec,
                                     jnp.int32(-2147483648)))

        for w in range(n_win):
            n0 = w * _WIN
            wn = min(_WIN, n_nodes - n0)

            @pl.when(c == (w % _NCORES))
            def _(w=w, n0=n0, wn=wn):
                r0 = extract(w)
                r1 = extract(w + 1)
                nfull, ztail = divmod(zrows, _CHUNK)
                for q in range(nfull):
                    pltpu.sync_copy(
                        zbuf_v,
                        win_sh.at[pl.ds(s * zrows + q * _CHUNK, _CHUNK)])
                if ztail:
                    pltpu.sync_copy(
                        zbuf_v.at[pl.ds(0, ztail)],
                        win_sh.at[pl.ds(s * zrows + nfull * _CHUNK, ztail)])
                plsc.subcore_barrier()

                k0 = r0 // _CHUNK
                k1 = (r1 + _CHUNK - 1) // _CHUNK
                nk = jnp.maximum(0, (k1 - k0 - s + _NSUB - 1) // _NSUB)
                odd = nk & 1

                @pl.when(odd == 1)
                def _(r0=r0, r1=r1, n0=n0):
                    kk = k0 + s
                    in_start(kk, 0)
                    in_wait(kk, 0)
                    process(kk, 0, r0, r1, n0)

                cb = k0 + s + odd * _NSUB
                rem = nk - odd

                @pl.when(rem > 0)
                def _(cb=cb):
                    in_start(cb, 0)

                def pair(t, carry, r0=r0, r1=r1, n0=n0, cb=cb, rem=rem):
                    for b in range(2):
                        q = 2 * t + b
                        kk = cb + q * _NSUB
                        in_wait(kk, b)

                        @pl.when(q + 1 < rem)
                        def _(q=q, b=b, t=t):
                            if b == 1:
                                sc_wait(0)
                            else:
                                @pl.when(t >= 1)
                                def _():
                                    sc_wait(1)
                            in_start(cb + (q + 1) * _NSUB, 1 - b)

                        compute_idx(kk, b, r0, r1, n0)
                        sc_start(b)
                    return carry

                lax.fori_loop(0, rem >> 1, pair, 0)

                @pl.when(rem >= 2)
                def _():
                    sc_wait(0)
                    sc_wait(1)

                plsc.subcore_barrier()

                share = (wn // _NSUB) & ~7
                if share > 0:
                    pltpu.sync_copy(
                        win_sh.at[pl.ds(s * share, share)],
                        out_hbm.at[pl.ds(n0 + s * share, share)])
                rem = wn - share * _NSUB
                if rem > 0:
                    @pl.when(s == 0)
                    def _(share=share, rem=rem, n0=n0):
                        pltpu.sync_copy(
                            win_sh.at[pl.ds(share * _NSUB, rem)],
                            out_hbm.at[pl.ds(n0 + share * _NSUB, rem)])
                plsc.subcore_barrier()

    return k(hs, seg, bounds)


def _tc_finish(pooled, degs2d, w0t, b0, w1t, b1):
    n, dout = pooled.shape
    dh = w0t.shape[1]
    bn = 2000
    assert n % bn == 0

    def body(p_r, d_r, w0_r, b0_r, w1_r, b1_r, o_r):
        f1 = jnp.maximum(d_r[...] * w0_r[...] + b0_r[...], 0.0)
        f = jnp.dot(f1, w1_r[...], preferred_element_type=jnp.float32)
        f = f + b1_r[...]
        o_r[...] = jnp.maximum(p_r[...] * f, 0.0)

    return pl.pallas_call(
        body,
        grid=(n // bn,),
        in_specs=[pl.BlockSpec((bn, dout), lambda i: (i, 0)),
                  pl.BlockSpec((bn, 1), lambda i: (i, 0)),
                  pl.BlockSpec((1, dh), lambda i: (0, 0)),
                  pl.BlockSpec((1, dh), lambda i: (0, 0)),
                  pl.BlockSpec((dh, dout), lambda i: (0, 0)),
                  pl.BlockSpec((1, dout), lambda i: (0, 0))],
        out_specs=pl.BlockSpec((bn, dout), lambda i: (i, 0)),
        out_shape=jax.ShapeDtypeStruct((n, dout), jnp.float32),
    )(pooled, degs2d, w0t, b0, w1t, b1)


def kernel(nfeat, perm_idx, perm_vals, pool_seg_ids, pool_vals, degs,
           weights, bias, W0, b0, W1, b1):
    n_nodes, d_in = nfeat.shape
    l = weights.shape[2]
    dout = weights.shape[1]
    m = perm_idx.shape[0]
    p = m // l

    packed = lax.bitcast_convert_type(nfeat.astype(jnp.bfloat16), jnp.int32)
    xi = _sc_gather(packed, perm_idx, perm_vals)

    eye8 = jnp.eye(8, dtype=jnp.float32)
    w0_big = (eye8[:, None, :, None]
              * jnp.transpose(weights[0])[None, :, None, :]).reshape(
                  8 * l, 8 * dout).astype(jnp.bfloat16)
    w1_big = (eye8[:, None, :, None]
              * jnp.transpose(weights[1])[None, :, None, :]).reshape(
                  8 * l, 8 * dout).astype(jnp.bfloat16)
    bias_wide = jnp.tile(bias, (1, 8))
    hs = _tc_matmul(xi, pool_vals.reshape(p, 1),
                    w0_big, w1_big, bias_wide, p, l, dout)

    n_win = -(-n_nodes // _WIN)
    cuts = jnp.arange(1, n_win + 1, dtype=jnp.int32) * _WIN
    bounds = jnp.searchsorted(pool_seg_ids, cuts, side="left").astype(jnp.int32)
    bounds = jnp.concatenate([jnp.zeros((1,), jnp.int32), bounds])
    bounds = jnp.pad(bounds, (0, 16 - bounds.shape[0]),
                     constant_values=jnp.int32(m // l))
    pooled = _sc_segsum(hs, pool_seg_ids, bounds, n_nodes)

    out = _tc_finish(pooled, degs[:, None], jnp.transpose(W0), b0[None, :],
                     jnp.transpose(W1), b1[None, :])
    return out
